# baseline (speedup 1.0000x reference)
import functools

import jax
import jax.numpy as jnp
from jax import lax
from jax.experimental import pallas as pl
from jax.experimental.pallas import tpu as pltpu
from jax.experimental.pallas import tpu_sc as plsc

NUM_CRITERIA = 26
NUM_MODELS = 100000
D = 32
BATCH = 16384

NUM_CORES = 2
NUM_SUBCORES = 16
NUM_WORKERS = NUM_CORES * NUM_SUBCORES
B_PER_W = BATCH // NUM_WORKERS
LANES = 16
CHUNKS = B_PER_W // LANES

_mesh = plsc.VectorSubcoreMesh(core_axis_name="c", subcore_axis_name="s")


@functools.partial(
    pl.kernel,
    out_type=jax.ShapeDtypeStruct((BATCH,), jnp.float32),
    mesh=_mesh,
    scratch_types=[
        pltpu.VMEM((B_PER_W,), jnp.int32),
        pltpu.VMEM((B_PER_W,), jnp.int32),
        pltpu.VMEM((B_PER_W,), jnp.int32),
        pltpu.VMEM((B_PER_W,), jnp.int32),
        pltpu.VMEM((B_PER_W, D), jnp.float32),
        pltpu.VMEM((B_PER_W, D), jnp.float32),
        pltpu.VMEM((B_PER_W, D), jnp.float32),
        pltpu.VMEM((B_PER_W,), jnp.float32),
        pltpu.VMEM((17 * LANES,), jnp.float32),
        pltpu.SemaphoreType.DMA,
    ],
    compiler_params=pltpu.CompilerParams(
        needs_layout_passes=False, use_tc_tiling_on_sc=False),
)
def _vbt_kernel(c_hbm, i_hbm, j_hbm, k_hbm, u_hbm, v_hbm, out_hbm,
                cidx_v, iidx_v, jidx_v, kidx_v, u_rows, vj_rows, vk_rows,
                out_v, tbuf, sem):
    wid = lax.axis_index("s") * NUM_CORES + lax.axis_index("c")
    base = wid * B_PER_W

    cp_c = pltpu.async_copy(c_hbm.at[pl.ds(base, B_PER_W)], cidx_v, sem)
    cp_i = pltpu.async_copy(i_hbm.at[pl.ds(base, B_PER_W)], iidx_v, sem)
    cp_j = pltpu.async_copy(j_hbm.at[pl.ds(base, B_PER_W)], jidx_v, sem)
    cp_k = pltpu.async_copy(k_hbm.at[pl.ds(base, B_PER_W)], kidx_v, sem)
    cp_c.wait()
    cp_i.wait()
    cp_j.wait()
    cp_k.wait()

    for c in range(CHUNKS):
        sl = pl.ds(c * LANES, LANES)
        iidx_v[sl] = cidx_v[sl] * NUM_MODELS + iidx_v[sl]

    g_u = pltpu.async_copy(u_hbm.at[iidx_v], u_rows, sem)
    g_j = pltpu.async_copy(v_hbm.at[jidx_v], vj_rows, sem)
    g_k = pltpu.async_copy(v_hbm.at[kidx_v], vk_rows, sem)
    g_u.wait()
    g_j.wait()
    g_k.wait()

    lane17 = lax.iota(jnp.int32, LANES) * 17
    one = jnp.full((LANES,), 1.0, jnp.float32)

    def chunk_body(c, carry):
        def elem_body(e, carry2):
            b = c * LANES + e
            u0 = u_rows[b, pl.ds(0, LANES)]
            u1 = u_rows[b, pl.ds(LANES, LANES)]
            d0 = vj_rows[b, pl.ds(0, LANES)] - vk_rows[b, pl.ds(0, LANES)]
            d1 = vj_rows[b, pl.ds(LANES, LANES)] - vk_rows[b, pl.ds(LANES, LANES)]
            p = u0 * d0 + u1 * d1
            plsc.store_scatter(tbuf, [lane17 + e], p)
            return carry2

        lax.fori_loop(0, LANES, elem_body, 0, unroll=4)

        acc = tbuf[pl.ds(0, LANES)]
        for l in range(1, LANES):
            acc = acc + tbuf[pl.ds(l * 17, LANES)]
        out_v[pl.ds(c * LANES, LANES)] = one / (one + jnp.exp(-acc))
        return carry

    lax.fori_loop(0, CHUNKS, chunk_body, 0)

    pltpu.sync_copy(out_v, out_hbm.at[pl.ds(base, B_PER_W)])


@jax.jit
def kernel(criterion_idx, i_idx, j_idx, k_idx, u_weight, v_weight):
    return _vbt_kernel(
        criterion_idx.astype(jnp.int32),
        i_idx.astype(jnp.int32),
        j_idx.astype(jnp.int32),
        k_idx.astype(jnp.int32),
        u_weight,
        v_weight,
    )


# speedup vs baseline: 1.6390x; 1.6390x over previous
import functools

import jax
import jax.numpy as jnp
from jax import lax
from jax.experimental import pallas as pl
from jax.experimental.pallas import tpu as pltpu
from jax.experimental.pallas import tpu_sc as plsc

NUM_CRITERIA = 26
NUM_MODELS = 100000
D = 32
BATCH = 16384

NUM_CORES = 2
NUM_SUBCORES = 16
NUM_WORKERS = NUM_CORES * NUM_SUBCORES
B_PER_W = BATCH // NUM_WORKERS
LANES = 16
CHUNK = 128
N_CHUNKS = B_PER_W // CHUNK
GROUPS = CHUNK // LANES

_mesh = plsc.VectorSubcoreMesh(core_axis_name="c", subcore_axis_name="s")


@functools.partial(
    pl.kernel,
    out_type=jax.ShapeDtypeStruct((BATCH,), jnp.float32),
    mesh=_mesh,
    scratch_types=[
        pltpu.VMEM((B_PER_W,), jnp.int32),
        pltpu.VMEM((B_PER_W,), jnp.int32),
        pltpu.VMEM((B_PER_W,), jnp.int32),
        pltpu.VMEM((B_PER_W,), jnp.int32),
        pltpu.VMEM((2, CHUNK, D), jnp.float32),
        pltpu.VMEM((2, CHUNK, D), jnp.float32),
        pltpu.VMEM((2, CHUNK, D), jnp.float32),
        pltpu.VMEM((B_PER_W,), jnp.float32),
        pltpu.VMEM((17 * LANES,), jnp.float32),
        pltpu.SemaphoreType.DMA,
        pltpu.SemaphoreType.DMA,
    ],
    compiler_params=pltpu.CompilerParams(
        needs_layout_passes=False, use_tc_tiling_on_sc=True),
)
def _vbt_kernel(c_hbm, i_hbm, j_hbm, k_hbm, u_hbm, v_hbm, out_hbm,
                cidx_v, iidx_v, jidx_v, kidx_v, u_rows, vj_rows, vk_rows,
                out_v, tbuf, sem_a, sem_b):
    wid = lax.axis_index("s") * NUM_CORES + lax.axis_index("c")
    base = wid * B_PER_W
    sems = (sem_a, sem_b)

    cp_c = pltpu.async_copy(c_hbm.at[pl.ds(base, B_PER_W)], cidx_v, sem_a)
    cp_i = pltpu.async_copy(i_hbm.at[pl.ds(base, B_PER_W)], iidx_v, sem_a)
    cp_j = pltpu.async_copy(j_hbm.at[pl.ds(base, B_PER_W)], jidx_v, sem_a)
    cp_k = pltpu.async_copy(k_hbm.at[pl.ds(base, B_PER_W)], kidx_v, sem_a)
    cp_c.wait()
    cp_i.wait()
    cp_j.wait()
    cp_k.wait()

    def flat_body(c, carry):
        sl = pl.ds(c * LANES, LANES)
        iidx_v[sl] = cidx_v[sl] * NUM_MODELS + iidx_v[sl]
        return carry

    lax.fori_loop(0, B_PER_W // LANES, flat_body, 0)

    def fire(c, buf):
        sem = sems[buf]

        def gbody(g, carry):
            off = c * CHUNK + g * LANES
            uvec = iidx_v[pl.ds(off, LANES)]
            jvec = jidx_v[pl.ds(off, LANES)]
            kvec = kidx_v[pl.ds(off, LANES)]
            for e in range(LANES):
                slot = g * LANES + e
                pltpu.async_copy(u_hbm.at[uvec[e]], u_rows.at[buf, slot], sem)
                pltpu.async_copy(v_hbm.at[jvec[e]], vj_rows.at[buf, slot], sem)
                pltpu.async_copy(v_hbm.at[kvec[e]], vk_rows.at[buf, slot], sem)
            return carry

        lax.fori_loop(0, GROUPS, gbody, 0)

    def drain(buf):
        sem = sems[buf]
        pltpu.make_async_copy(u_hbm.at[pl.ds(0, CHUNK)],
                              u_rows.at[buf], sem).wait()
        pltpu.make_async_copy(v_hbm.at[pl.ds(0, CHUNK)],
                              vj_rows.at[buf], sem).wait()
        pltpu.make_async_copy(v_hbm.at[pl.ds(0, CHUNK)],
                              vk_rows.at[buf], sem).wait()

    lane17 = lax.iota(jnp.int32, LANES) * 17
    one = jnp.full((LANES,), 1.0, jnp.float32)

    def compute(c, buf):
        def cbody(g, carry):
            for e in range(LANES):
                b = g * LANES + e
                u0 = u_rows[buf, b, pl.ds(0, LANES)]
                u1 = u_rows[buf, b, pl.ds(LANES, LANES)]
                d0 = (vj_rows[buf, b, pl.ds(0, LANES)]
                      - vk_rows[buf, b, pl.ds(0, LANES)])
                d1 = (vj_rows[buf, b, pl.ds(LANES, LANES)]
                      - vk_rows[buf, b, pl.ds(LANES, LANES)])
                p = u0 * d0 + u1 * d1
                plsc.store_scatter(tbuf, [lane17 + e], p)
            acc = tbuf[pl.ds(0, LANES)]
            for l in range(1, LANES):
                acc = acc + tbuf[pl.ds(l * 17, LANES)]
            out_v[pl.ds(c * CHUNK + g * LANES, LANES)] = (
                one / (one + jnp.exp(-acc)))
            return carry

        lax.fori_loop(0, GROUPS, cbody, 0)

    fire(0, 0)
    for c in range(N_CHUNKS):
        if c + 1 < N_CHUNKS:
            fire(c + 1, (c + 1) % 2)
        drain(c % 2)
        compute(c, c % 2)

    pltpu.sync_copy(out_v, out_hbm.at[pl.ds(base, B_PER_W)])


@jax.jit
def kernel(criterion_idx, i_idx, j_idx, k_idx, u_weight, v_weight):
    return _vbt_kernel(
        criterion_idx.astype(jnp.int32),
        i_idx.astype(jnp.int32),
        j_idx.astype(jnp.int32),
        k_idx.astype(jnp.int32),
        u_weight,
        v_weight,
    )
